# baseline (speedup 1.0000x reference)
import functools

import jax
import jax.numpy as jnp
from jax import lax
from jax.experimental import pallas as pl
from jax.experimental.pallas import tpu as pltpu
from jax.experimental.pallas import tpu_sc as plsc

NC = 2
NS = 16
NW = NC * NS
CHUNK = 128


def _build_sc_agg(n, n_pad, d, cpw):
    rps = n_pad // NS
    assert cpw % 3 == 0 and cpw >= 6
    assert rps % 8 == 0

    @functools.partial(
        pl.kernel,
        mesh=plsc.VectorSubcoreMesh(core_axis_name="c", subcore_axis_name="s"),
        out_type=jax.ShapeDtypeStruct((NC, n_pad, d), jnp.float32),
        scratch_types=[
            pltpu.VMEM((3, 2, CHUNK), jnp.int32),
            pltpu.VMEM((3, CHUNK, d), jnp.float32),
            pltpu.VMEM_SHARED((n_pad, d), jnp.float32),
            pltpu.SemaphoreType.DMA,
            pltpu.SemaphoreType.DMA,
            pltpu.SemaphoreType.DMA,
            pltpu.SemaphoreType.DMA,
            pltpu.SemaphoreType.DMA,
            pltpu.SemaphoreType.DMA,
            pltpu.SemaphoreType.DMA,
            pltpu.SemaphoreType.DMA,
            pltpu.SemaphoreType.DMA,
        ],
    )
    def sc_agg(h_hbm, idxc_hbm, out_hbm, idx_v, rows_v, acc, *sems):
        c = lax.axis_index("c")
        s = lax.axis_index("s")
        wid = s * NC + c
        isems = sems[0:3]
        gsems = sems[3:6]
        ssems = sems[6:9]

        def idx_start(k, m):
            pltpu.async_copy(idxc_hbm.at[wid, k], idx_v.at[m], isems[m])

        def idx_wait(m):
            pltpu.make_async_copy(idxc_hbm.at[wid, 0], idx_v.at[m],
                                  isems[m]).wait()

        def gather_start(m):
            pltpu.async_copy(h_hbm.at[idx_v.at[m, 0]], rows_v.at[m], gsems[m])

        def gather_wait(m):
            pltpu.make_async_copy(h_hbm.at[idx_v.at[m, 0]], rows_v.at[m],
                                  gsems[m]).wait()

        def scatter_start(m):
            pltpu.async_copy(rows_v.at[m], acc.at[idx_v.at[m, 1]], ssems[m],
                             add=True)

        def scatter_wait(m):
            pltpu.make_async_copy(rows_v.at[m], acc.at[idx_v.at[m, 1]],
                                  ssems[m]).wait()

        def zrow(r, carry):
            for j in range(d // 16):
                rows_v[0, r, pl.ds(j * 16, 16)] = jnp.zeros((16,), jnp.float32)
            return carry

        lax.fori_loop(0, CHUNK, zrow, 0)
        full, rem = divmod(rps, CHUNK)
        for q in range(full):
            pltpu.sync_copy(rows_v.at[0],
                            acc.at[pl.ds(s * rps + q * CHUNK, CHUNK)])
        if rem:
            pltpu.sync_copy(rows_v.at[0, pl.ds(0, rem)],
                            acc.at[pl.ds(s * rps + full * CHUNK, rem)])
        plsc.subcore_barrier()

        idx_start(0, 0)
        idx_start(1, 1)
        idx_wait(0)
        gather_start(0)
        idx_start(2, 2)
        idx_wait(1)
        gather_start(1)
        gather_wait(0)
        scatter_start(0)

        def step(p, carry):
            for j in range(3):
                k = 1 + 3 * p + j
                mk = (1 + j) % 3
                m1 = (2 + j) % 3
                m2 = j
                scatter_wait(m2)
                idx_start(k + 2, m2)
                idx_wait(m1)
                gather_start(m1)
                gather_wait(mk)
                scatter_start(mk)
            return carry

        lax.fori_loop(0, (cpw - 3) // 3, step, 0)
        mk = (cpw - 2) % 3
        m1 = (cpw - 1) % 3
        m2 = cpw % 3
        scatter_wait(m2)
        idx_wait(m1)
        gather_start(m1)
        gather_wait(mk)
        scatter_start(mk)
        gather_wait(m1)
        scatter_start(m1)
        scatter_wait(mk)
        scatter_wait(m1)

        plsc.subcore_barrier()
        pltpu.sync_copy(acc.at[pl.ds(s * rps, rps)],
                        out_hbm.at[c, pl.ds(s * rps, rps)])

    return sc_agg


def _tc_init_body(x_ref, wp_ref, bp_ref, out_ref):
    out_ref[...] = (
        jnp.dot(x_ref[...], wp_ref[...], preferred_element_type=jnp.float32)
        + bp_ref[...])


def _tc_pre_body(h_ref, w1_ref, b1_ref, out_ref):
    out_ref[...] = (
        jnp.dot(h_ref[...], w1_ref[...], preferred_element_type=jnp.float32)
        + b1_ref[...])


def _tc_post_body(h_ref, p1_ref, agg_ref, w1_ref, w2_ref, b2_ref, g_ref,
                  bb_ref, out_ref):
    h = h_ref[...]
    n = h.shape[0]
    aggsum = agg_ref[0, :n] + agg_ref[1, :n]
    u = jnp.maximum(
        p1_ref[...]
        + jnp.dot(aggsum, w1_ref[...], preferred_element_type=jnp.float32),
        0.0)
    u = (jnp.dot(u, w2_ref[...], preferred_element_type=jnp.float32)
         + b2_ref[...])
    mean = jnp.mean(u, axis=0, keepdims=True)
    var = jnp.mean((u - mean) ** 2, axis=0, keepdims=True)
    un = (u - mean) / jnp.sqrt(var + 1e-5) * g_ref[...] + bb_ref[...]
    out_ref[...] = jnp.maximum(un, 0.0) + h


def _tc_head_body(h_ref, batch_ref, w1_ref, b1_ref, w2t_ref, b2_ref, out_ref):
    n = h_ref.shape[0]
    g = out_ref.shape[0]
    gids = lax.broadcasted_iota(jnp.int32, (g, n), 0)
    onehot = (batch_ref[...] == gids).astype(jnp.float32)
    pooled = jnp.dot(onehot, h_ref[...], preferred_element_type=jnp.float32)
    o1 = jnp.maximum(
        jnp.dot(pooled, w1_ref[...], preferred_element_type=jnp.float32)
        + b1_ref[...], 0.0)
    out_ref[...] = jnp.sum(o1 * w2t_ref[...], axis=1, keepdims=True) + b2_ref[...]


def kernel(x, edge_index, edge_attr, batch, Wp, bp, mlp1_w, mlp1_b, mlp2_w,
           mlp2_b, bn_g, bn_b, out1_w, out1_b, out2_w, out2_b):
    n, d = x.shape
    e = edge_index.shape[1]
    num_layers = mlp1_w.shape[0]
    h2 = mlp1_w.shape[2]
    num_graphs = 64

    cpw = -(-e // (NW * CHUNK))
    cpw += (-cpw) % 3
    e_pad = NW * cpw * CHUNK
    total_chunks = NW * cpw
    n_pad = ((n + NS * 8 - 1) // (NS * 8)) * (NS * 8)
    if n_pad == n:
        n_pad += NS * 8

    src = edge_index[0].astype(jnp.int32)
    dst = edge_index[1].astype(jnp.int32)
    pad = e_pad - e
    pidx = jnp.arange(pad, dtype=jnp.int32)
    src_c = jnp.concatenate([src, pidx % n]).reshape(NW, cpw, CHUNK)
    dst_c = jnp.concatenate([dst, n + pidx % (n_pad - n)]).reshape(
        NW, cpw, CHUNK)
    idx_c = jnp.stack([src_c, dst_c], axis=2)
    del total_chunks

    sc_agg = _build_sc_agg(n, n_pad, d, cpw)

    tc_init = pl.pallas_call(
        _tc_init_body, out_shape=jax.ShapeDtypeStruct((n, d), jnp.float32))
    tc_pre = pl.pallas_call(
        _tc_pre_body, out_shape=jax.ShapeDtypeStruct((n, h2), jnp.float32))
    tc_post = pl.pallas_call(
        _tc_post_body, out_shape=jax.ShapeDtypeStruct((n, d), jnp.float32))
    tc_head = pl.pallas_call(
        _tc_head_body,
        out_shape=jax.ShapeDtypeStruct((num_graphs, 1), jnp.float32))

    h = tc_init(x, Wp, bp.reshape(1, d))
    for i in range(num_layers):
        agg = sc_agg(h, idx_c)
        p1 = tc_pre(h, mlp1_w[i], mlp1_b[i].reshape(1, h2))
        h = tc_post(h, p1, agg, mlp1_w[i], mlp2_w[i],
                    mlp2_b[i].reshape(1, d), bn_g[i].reshape(1, d),
                    bn_b[i].reshape(1, d))
    o = tc_head(h, batch.reshape(1, n), out1_w, out1_b.reshape(1, d),
                out2_w.reshape(1, d), out2_b.reshape(1, 1))
    return o.reshape(num_graphs)


# speedup vs baseline: 1.0286x; 1.0286x over previous
import functools

import jax
import jax.numpy as jnp
from jax import lax
from jax.experimental import pallas as pl
from jax.experimental.pallas import tpu as pltpu
from jax.experimental.pallas import tpu_sc as plsc

NC = 2
NS = 16
NW = NC * NS
CHUNK = 128


def _build_sc_agg(n, n_pad, d, cpw):
    rps = n_pad // NS
    assert cpw >= 6
    assert rps % 8 == 0

    @functools.partial(
        pl.kernel,
        mesh=plsc.VectorSubcoreMesh(core_axis_name="c", subcore_axis_name="s"),
        out_type=jax.ShapeDtypeStruct((NC, n_pad, d), jnp.float32),
        scratch_types=[
            pltpu.VMEM((3, 2, CHUNK), jnp.int32),
            pltpu.VMEM((3, CHUNK, d), jnp.float32),
            pltpu.VMEM_SHARED((n_pad, d), jnp.float32),
            pltpu.SemaphoreType.DMA,
            pltpu.SemaphoreType.DMA,
            pltpu.SemaphoreType.DMA,
            pltpu.SemaphoreType.DMA,
            pltpu.SemaphoreType.DMA,
            pltpu.SemaphoreType.DMA,
            pltpu.SemaphoreType.DMA,
            pltpu.SemaphoreType.DMA,
            pltpu.SemaphoreType.DMA,
        ],
    )
    def sc_agg(h_hbm, idxc_hbm, out_hbm, idx_v, rows_v, acc, *sems):
        c = lax.axis_index("c")
        s = lax.axis_index("s")
        wid = s * NC + c
        isems = sems[0:3]
        gsems = sems[3:6]
        ssems = sems[6:9]

        def idx_start(k, m):
            pltpu.async_copy(idxc_hbm.at[wid, k], idx_v.at[m], isems[m])

        def idx_wait(m):
            pltpu.make_async_copy(idxc_hbm.at[wid, 0], idx_v.at[m],
                                  isems[m]).wait()

        def gather_start(m):
            pltpu.async_copy(h_hbm.at[idx_v.at[m, 0]], rows_v.at[m], gsems[m])

        def gather_wait(m):
            pltpu.make_async_copy(h_hbm.at[idx_v.at[m, 0]], rows_v.at[m],
                                  gsems[m]).wait()

        def scatter_start(m):
            pltpu.async_copy(rows_v.at[m], acc.at[idx_v.at[m, 1]], ssems[m],
                             add=True)

        def scatter_wait(m):
            pltpu.make_async_copy(rows_v.at[m], acc.at[idx_v.at[m, 1]],
                                  ssems[m]).wait()

        def zrow(r, carry):
            for j in range(d // 16):
                rows_v[0, r, pl.ds(j * 16, 16)] = jnp.zeros((16,), jnp.float32)
            return carry

        lax.fori_loop(0, CHUNK, zrow, 0)
        full, rem = divmod(rps, CHUNK)
        for q in range(full):
            pltpu.sync_copy(rows_v.at[0],
                            acc.at[pl.ds(s * rps + q * CHUNK, CHUNK)])
        if rem:
            pltpu.sync_copy(rows_v.at[0, pl.ds(0, rem)],
                            acc.at[pl.ds(s * rps + full * CHUNK, rem)])
        plsc.subcore_barrier()

        idx_start(0, 0)
        idx_start(1, 1)
        idx_wait(0)
        gather_start(0)
        idx_start(2, 2)
        idx_wait(1)
        gather_start(1)
        gather_wait(0)
        scatter_start(0)

        def step(p, carry):
            for j in range(3):
                k = 1 + 3 * p + j
                mk = (1 + j) % 3
                m1 = (2 + j) % 3
                m2 = j
                scatter_wait(m2)
                idx_start(k + 2, m2)
                idx_wait(m1)
                gather_start(m1)
                gather_wait(mk)
                scatter_start(mk)
            return carry

        trip = (cpw - 3) // 3
        lax.fori_loop(0, trip, step, 0)
        for k in range(1 + 3 * trip, cpw - 2):
            mk, m1, m2 = k % 3, (k + 1) % 3, (k + 2) % 3
            scatter_wait(m2)
            idx_start(k + 2, m2)
            idx_wait(m1)
            gather_start(m1)
            gather_wait(mk)
            scatter_start(mk)
        mk = (cpw - 2) % 3
        m1 = (cpw - 1) % 3
        m2 = cpw % 3
        scatter_wait(m2)
        idx_wait(m1)
        gather_start(m1)
        gather_wait(mk)
        scatter_start(mk)
        gather_wait(m1)
        scatter_start(m1)
        scatter_wait(mk)
        scatter_wait(m1)

        plsc.subcore_barrier()
        pltpu.sync_copy(acc.at[pl.ds(s * rps, rps)],
                        out_hbm.at[c, pl.ds(s * rps, rps)])

    return sc_agg


def _tc_init_body(x_ref, wp_ref, bp_ref, out_ref):
    out_ref[...] = (
        jnp.dot(x_ref[...], wp_ref[...], preferred_element_type=jnp.float32)
        + bp_ref[...])


def _tc_layer_a_body(h_ref, agg_ref, w1_ref, b1_ref, w2_ref, b2_ref, u_ref,
                     st_ref):
    t = h_ref[...] + agg_ref[0] + agg_ref[1]
    u = jnp.maximum(
        jnp.dot(t, w1_ref[...], preferred_element_type=jnp.float32)
        + b1_ref[...], 0.0)
    u = (jnp.dot(u, w2_ref[...], preferred_element_type=jnp.float32)
         + b2_ref[...])
    u_ref[...] = u
    su = jnp.sum(u, axis=0, keepdims=True)
    s2 = jnp.sum(u * u, axis=0, keepdims=True)
    st_ref[0] = jnp.concatenate(
        [su, s2, jnp.zeros((6, u.shape[1]), jnp.float32)], axis=0)


def _tc_layer_b_body(u_ref, h_ref, st_ref, g_ref, bb_ref, out_ref, *, n):
    st = st_ref[...]
    mean = jnp.sum(st[:, 0, :], axis=0, keepdims=True) / n
    var = jnp.sum(st[:, 1, :], axis=0, keepdims=True) / n - mean * mean
    u = u_ref[...]
    un = (u - mean) / jnp.sqrt(var + 1e-5) * g_ref[...] + bb_ref[...]
    out_ref[...] = jnp.maximum(un, 0.0) + h_ref[...]


def _tc_head_body(h_ref, batch_ref, w1_ref, b1_ref, w2t_ref, b2_ref, out_ref):
    n = h_ref.shape[0]
    g = out_ref.shape[0]
    gids = lax.broadcasted_iota(jnp.int32, (g, n), 0)
    onehot = (batch_ref[...] == gids).astype(jnp.float32)
    pooled = jnp.dot(onehot, h_ref[...], preferred_element_type=jnp.float32)
    o1 = jnp.maximum(
        jnp.dot(pooled, w1_ref[...], preferred_element_type=jnp.float32)
        + b1_ref[...], 0.0)
    out_ref[...] = jnp.sum(o1 * w2t_ref[...], axis=1, keepdims=True) + b2_ref[...]


def kernel(x, edge_index, edge_attr, batch, Wp, bp, mlp1_w, mlp1_b, mlp2_w,
           mlp2_b, bn_g, bn_b, out1_w, out1_b, out2_w, out2_b):
    n, d = x.shape
    e = edge_index.shape[1]
    num_layers = mlp1_w.shape[0]
    h2 = mlp1_w.shape[2]
    num_graphs = 64

    cpw = -(-e // (NW * CHUNK))
    e_pad = NW * cpw * CHUNK
    total_chunks = NW * cpw
    n_pad = ((n + NS * 8 - 1) // (NS * 8)) * (NS * 8)
    if n_pad == n:
        n_pad += NS * 8

    src = edge_index[0].astype(jnp.int32)
    dst = edge_index[1].astype(jnp.int32)
    pad = e_pad - e
    pidx = jnp.arange(pad, dtype=jnp.int32)
    src_c = jnp.concatenate([src, pidx % n]).reshape(NW, cpw, CHUNK)
    dst_c = jnp.concatenate([dst, n + pidx % (n_pad - n)]).reshape(
        NW, cpw, CHUNK)
    idx_c = jnp.stack([src_c, dst_c], axis=2)
    del total_chunks

    sc_agg = _build_sc_agg(n, n_pad, d, cpw)

    bs = 2000
    nb = n // bs
    assert nb * bs == n

    def _row(i):
        return (i, 0)

    def _fixed2(i):
        return (0, 0)

    tc_init = pl.pallas_call(
        _tc_init_body,
        grid=(nb,),
        in_specs=[
            pl.BlockSpec((bs, d), _row),
            pl.BlockSpec((d, d), _fixed2),
            pl.BlockSpec((1, d), _fixed2),
        ],
        out_specs=pl.BlockSpec((bs, d), _row),
        out_shape=jax.ShapeDtypeStruct((n, d), jnp.float32))
    tc_layer_a = pl.pallas_call(
        _tc_layer_a_body,
        grid=(nb,),
        in_specs=[
            pl.BlockSpec((bs, d), _row),
            pl.BlockSpec((NC, bs, d), lambda i: (0, i, 0)),
            pl.BlockSpec((d, h2), _fixed2),
            pl.BlockSpec((1, h2), _fixed2),
            pl.BlockSpec((h2, d), _fixed2),
            pl.BlockSpec((1, d), _fixed2),
        ],
        out_specs=[
            pl.BlockSpec((bs, d), _row),
            pl.BlockSpec((1, 8, d), lambda i: (i, 0, 0)),
        ],
        out_shape=[
            jax.ShapeDtypeStruct((n, d), jnp.float32),
            jax.ShapeDtypeStruct((nb, 8, d), jnp.float32),
        ])
    tc_layer_b = pl.pallas_call(
        functools.partial(_tc_layer_b_body, n=float(n)),
        grid=(nb,),
        in_specs=[
            pl.BlockSpec((bs, d), _row),
            pl.BlockSpec((bs, d), _row),
            pl.BlockSpec((nb, 8, d), lambda i: (0, 0, 0)),
            pl.BlockSpec((1, d), _fixed2),
            pl.BlockSpec((1, d), _fixed2),
        ],
        out_specs=pl.BlockSpec((bs, d), _row),
        out_shape=jax.ShapeDtypeStruct((n, d), jnp.float32))
    tc_head = pl.pallas_call(
        _tc_head_body,
        out_shape=jax.ShapeDtypeStruct((num_graphs, 1), jnp.float32))

    h = tc_init(x, Wp, bp.reshape(1, d))
    for i in range(num_layers):
        agg = sc_agg(h, idx_c)
        u, st = tc_layer_a(h, agg, mlp1_w[i], mlp1_b[i].reshape(1, h2),
                           mlp2_w[i], mlp2_b[i].reshape(1, d))
        h = tc_layer_b(u, h, st, bn_g[i].reshape(1, d), bn_b[i].reshape(1, d))
    o = tc_head(h, batch.reshape(1, n), out1_w, out1_b.reshape(1, d),
                out2_w.reshape(1, d), out2_b.reshape(1, 1))
    return o.reshape(num_graphs)


# speedup vs baseline: 1.0568x; 1.0274x over previous
import functools

import jax
import jax.numpy as jnp
from jax import lax
from jax.experimental import pallas as pl
from jax.experimental.pallas import tpu as pltpu
from jax.experimental.pallas import tpu_sc as plsc

NC = 2
NS = 16
NW = NC * NS
CHUNK = 128


def _build_sc_agg(n, n_pad, d, cpw):
    rps = n_pad // NS
    assert cpw >= 6
    assert rps % 8 == 0

    @functools.partial(
        pl.kernel,
        mesh=plsc.VectorSubcoreMesh(core_axis_name="c", subcore_axis_name="s"),
        out_type=jax.ShapeDtypeStruct((NC, n_pad, d), jnp.float32),
        scratch_types=[
            pltpu.VMEM((3, 2, CHUNK), jnp.int32),
            pltpu.VMEM((3, CHUNK, d), jnp.float32),
            pltpu.VMEM_SHARED((n_pad, d), jnp.float32),
            pltpu.SemaphoreType.DMA,
            pltpu.SemaphoreType.DMA,
            pltpu.SemaphoreType.DMA,
            pltpu.SemaphoreType.DMA,
            pltpu.SemaphoreType.DMA,
            pltpu.SemaphoreType.DMA,
            pltpu.SemaphoreType.DMA,
            pltpu.SemaphoreType.DMA,
            pltpu.SemaphoreType.DMA,
        ],
    )
    def sc_agg(h_hbm, idxc_hbm, out_hbm, idx_v, rows_v, acc, *sems):
        c = lax.axis_index("c")
        s = lax.axis_index("s")
        wid = s * NC + c
        isems = sems[0:3]
        gsems = sems[3:6]
        ssems = sems[6:9]

        def idx_start(k, m):
            pltpu.async_copy(idxc_hbm.at[wid, k], idx_v.at[m], isems[m])

        def idx_wait(m):
            pltpu.make_async_copy(idxc_hbm.at[wid, 0], idx_v.at[m],
                                  isems[m]).wait()

        def gather_start(m):
            pltpu.async_copy(h_hbm.at[idx_v.at[m, 0]], rows_v.at[m], gsems[m])

        def gather_wait(m):
            pltpu.make_async_copy(h_hbm.at[idx_v.at[m, 0]], rows_v.at[m],
                                  gsems[m]).wait()

        def scatter_start(m):
            pltpu.async_copy(rows_v.at[m], acc.at[idx_v.at[m, 1]], ssems[m],
                             add=True)

        def scatter_wait(m):
            pltpu.make_async_copy(rows_v.at[m], acc.at[idx_v.at[m, 1]],
                                  ssems[m]).wait()

        def zrow(r, carry):
            for j in range(d // 16):
                rows_v[0, r, pl.ds(j * 16, 16)] = jnp.zeros((16,), jnp.float32)
            return carry

        lax.fori_loop(0, CHUNK, zrow, 0)
        full, rem = divmod(rps, CHUNK)
        for q in range(full):
            pltpu.sync_copy(rows_v.at[0],
                            acc.at[pl.ds(s * rps + q * CHUNK, CHUNK)])
        if rem:
            pltpu.sync_copy(rows_v.at[0, pl.ds(0, rem)],
                            acc.at[pl.ds(s * rps + full * CHUNK, rem)])
        plsc.subcore_barrier()

        idx_start(0, 0)
        idx_start(1, 1)
        idx_wait(0)
        gather_start(0)
        idx_start(2, 2)
        idx_wait(1)
        gather_start(1)
        gather_wait(0)
        scatter_start(0)

        def step(p, carry):
            for j in range(3):
                k = 1 + 3 * p + j
                mk = (1 + j) % 3
                m1 = (2 + j) % 3
                m2 = j
                scatter_wait(m2)
                idx_start(k + 2, m2)
                idx_wait(m1)
                gather_start(m1)
                gather_wait(mk)
                scatter_start(mk)
            return carry

        trip = (cpw - 3) // 3
        lax.fori_loop(0, trip, step, 0)
        for k in range(1 + 3 * trip, cpw - 2):
            mk, m1, m2 = k % 3, (k + 1) % 3, (k + 2) % 3
            scatter_wait(m2)
            idx_start(k + 2, m2)
            idx_wait(m1)
            gather_start(m1)
            gather_wait(mk)
            scatter_start(mk)
        mk = (cpw - 2) % 3
        m1 = (cpw - 1) % 3
        m2 = cpw % 3
        scatter_wait(m2)
        idx_wait(m1)
        gather_start(m1)
        gather_wait(mk)
        scatter_start(mk)
        gather_wait(m1)
        scatter_start(m1)
        scatter_wait(mk)
        scatter_wait(m1)

        plsc.subcore_barrier()
        pltpu.sync_copy(acc.at[pl.ds(s * rps, rps)],
                        out_hbm.at[c, pl.ds(s * rps, rps)])

    return sc_agg


def _tc_init_body(x_ref, wp_ref, bp_ref, out_ref):
    out_ref[...] = (
        jnp.dot(x_ref[...], wp_ref[...], preferred_element_type=jnp.float32)
        + bp_ref[...])


def _tc_layer_body(h_ref, agg_ref, w1_ref, b1_ref, w2_ref, b2_ref, g_ref,
                   bb_ref, out_ref):
    h = h_ref[...]
    n = h.shape[0]
    t = h + agg_ref[0, :n] + agg_ref[1, :n]
    u = jnp.maximum(
        jnp.dot(t, w1_ref[...], preferred_element_type=jnp.float32)
        + b1_ref[...], 0.0)
    u = (jnp.dot(u, w2_ref[...], preferred_element_type=jnp.float32)
         + b2_ref[...])
    mean = jnp.mean(u, axis=0, keepdims=True)
    var = jnp.mean((u - mean) ** 2, axis=0, keepdims=True)
    un = (u - mean) / jnp.sqrt(var + 1e-5) * g_ref[...] + bb_ref[...]
    out_ref[...] = jnp.maximum(un, 0.0) + h


def _tc_head_body(h_ref, batch_ref, w1_ref, b1_ref, w2t_ref, b2_ref, out_ref):
    n = h_ref.shape[0]
    g = out_ref.shape[0]
    gids = lax.broadcasted_iota(jnp.int32, (g, n), 0)
    onehot = (batch_ref[...] == gids).astype(jnp.float32)
    pooled = jnp.dot(onehot, h_ref[...], preferred_element_type=jnp.float32)
    o1 = jnp.maximum(
        jnp.dot(pooled, w1_ref[...], preferred_element_type=jnp.float32)
        + b1_ref[...], 0.0)
    out_ref[...] = jnp.sum(o1 * w2t_ref[...], axis=1, keepdims=True) + b2_ref[...]


def kernel(x, edge_index, edge_attr, batch, Wp, bp, mlp1_w, mlp1_b, mlp2_w,
           mlp2_b, bn_g, bn_b, out1_w, out1_b, out2_w, out2_b):
    n, d = x.shape
    e = edge_index.shape[1]
    num_layers = mlp1_w.shape[0]
    h2 = mlp1_w.shape[2]
    num_graphs = 64

    cpw = -(-e // (NW * CHUNK))
    e_pad = NW * cpw * CHUNK
    total_chunks = NW * cpw
    n_pad = ((n + NS * 8 - 1) // (NS * 8)) * (NS * 8)
    if n_pad == n:
        n_pad += NS * 8

    src = edge_index[0].astype(jnp.int32)
    dst = edge_index[1].astype(jnp.int32)
    pad = e_pad - e
    pidx = jnp.arange(pad, dtype=jnp.int32)
    src_c = jnp.concatenate([src, pidx % n]).reshape(NW, cpw, CHUNK)
    dst_c = jnp.concatenate([dst, n + pidx % (n_pad - n)]).reshape(
        NW, cpw, CHUNK)
    idx_c = jnp.stack([src_c, dst_c], axis=2)
    del total_chunks

    sc_agg = _build_sc_agg(n, n_pad, d, cpw)

    bs = 2000
    nb = n // bs
    assert nb * bs == n

    def _row(i):
        return (i, 0)

    def _fixed2(i):
        return (0, 0)

    tc_init = pl.pallas_call(
        _tc_init_body,
        grid=(nb,),
        in_specs=[
            pl.BlockSpec((bs, d), _row),
            pl.BlockSpec((d, d), _fixed2),
            pl.BlockSpec((1, d), _fixed2),
        ],
        out_specs=pl.BlockSpec((bs, d), _row),
        out_shape=jax.ShapeDtypeStruct((n, d), jnp.float32))
    tc_layer = pl.pallas_call(
        _tc_layer_body, out_shape=jax.ShapeDtypeStruct((n, d), jnp.float32))
    tc_head = pl.pallas_call(
        _tc_head_body,
        out_shape=jax.ShapeDtypeStruct((num_graphs, 1), jnp.float32))

    h = tc_init(x, Wp, bp.reshape(1, d))
    for i in range(num_layers):
        agg = sc_agg(h, idx_c)
        h = tc_layer(h, agg, mlp1_w[i], mlp1_b[i].reshape(1, h2), mlp2_w[i],
                     mlp2_b[i].reshape(1, d), bn_g[i].reshape(1, d),
                     bn_b[i].reshape(1, d))
    o = tc_head(h, batch.reshape(1, n), out1_w, out1_b.reshape(1, d),
                out2_w.reshape(1, d), out2_b.reshape(1, 1))
    return o.reshape(num_graphs)
